# baseline (speedup 1.0000x reference)
import functools

import jax
import jax.numpy as jnp
from jax import lax
from jax.experimental import pallas as pl
from jax.experimental.pallas import tpu as pltpu
from jax.experimental.pallas import tpu_sc as plsc

N = 10000
F0 = 128
F1 = 16
F2 = 2
E = 320000

NPAD = 10240
EW = 125
ROWS = E // EW

_NC = 2
_NS = 16
_RPT = ROWS // (_NC * _NS)
_SL = NPAD // _NS

_BR = 1024
_NB = 8


_MESH = plsc.VectorSubcoreMesh(core_axis_name="c", subcore_axis_name="s")
_SC_PARAMS = pltpu.CompilerParams(use_tc_tiling_on_sc=False)


def _make_segsum(width):
    @functools.partial(
        pl.kernel,
        out_type=jax.ShapeDtypeStruct((_NC, NPAD, width), jnp.float32),
        mesh=_MESH,
        compiler_params=_SC_PARAMS,
        scratch_types=[
            pltpu.VMEM((_RPT, EW), jnp.int32),
            pltpu.VMEM((_RPT, EW), jnp.int32),
            pltpu.VMEM((_NB, EW, width), jnp.float32),
            pltpu.VMEM_SHARED((NPAD, width), jnp.float32),
            pltpu.VMEM_SHARED((NPAD, width), jnp.float32),
            pltpu.SemaphoreType.DMA,
            pltpu.SemaphoreType.DMA,
        ],
    )
    def segsum(vals_hbm, src_hbm, dst_hbm, zeros_hbm, out_hbm,
               src_v, dst_v, rows_v, vals_sh, acc_sh, gsem, ssem):
        c = lax.axis_index("c")
        s = lax.axis_index("s")
        wid = s * _NC + c
        pltpu.sync_copy(src_hbm.at[pl.ds(wid * _RPT, _RPT)], src_v)
        pltpu.sync_copy(dst_hbm.at[pl.ds(wid * _RPT, _RPT)], dst_v)
        pltpu.sync_copy(vals_hbm.at[pl.ds(s * _SL, _SL)],
                        vals_sh.at[pl.ds(s * _SL, _SL)])
        pltpu.sync_copy(zeros_hbm.at[pl.ds(s * _SL, _SL)],
                        acc_sh.at[pl.ds(s * _SL, _SL)])
        plsc.subcore_barrier()

        @pl.loop(0, _RPT, step=_NB)
        def _(j0):
            gs = [pltpu.async_copy(vals_sh.at[src_v.at[j0 + b]],
                                   rows_v.at[b], gsem)
                  for b in range(_NB)]
            ss = []
            for b in range(_NB):
                gs[b].wait()
                ss.append(pltpu.async_copy(rows_v.at[b],
                                           acc_sh.at[dst_v.at[j0 + b]],
                                           ssem, add=True))
            for h in ss:
                h.wait()

        plsc.subcore_barrier()
        pltpu.sync_copy(acc_sh.at[pl.ds(s * _SL, _SL)],
                        out_hbm.at[c, pl.ds(s * _SL, _SL)])

    return segsum


def _make_degree():
    @functools.partial(
        pl.kernel,
        out_type=jax.ShapeDtypeStruct((_NC, NPAD, F1), jnp.float32),
        mesh=_MESH,
        compiler_params=_SC_PARAMS,
        scratch_types=[
            pltpu.VMEM((_RPT, EW), jnp.int32),
            pltpu.VMEM((EW, F1), jnp.float32),
            pltpu.VMEM_SHARED((NPAD, F1), jnp.float32),
            pltpu.SemaphoreType.DMA,
        ],
    )
    def degree(ones_hbm, dst_hbm, zeros_hbm, out_hbm,
               dst_v, ones_v, acc_sh, ssem):
        c = lax.axis_index("c")
        s = lax.axis_index("s")
        wid = s * _NC + c
        pltpu.sync_copy(dst_hbm.at[pl.ds(wid * _RPT, _RPT)], dst_v)
        pltpu.sync_copy(ones_hbm.at[pl.ds(0, EW)], ones_v)
        pltpu.sync_copy(zeros_hbm.at[pl.ds(s * _SL, _SL)],
                        acc_sh.at[pl.ds(s * _SL, _SL)])
        plsc.subcore_barrier()

        @pl.loop(0, _RPT, step=_NB)
        def _(j0):
            ss = [pltpu.async_copy(ones_v, acc_sh.at[dst_v.at[j0 + b]],
                                   ssem, add=True)
                  for b in range(_NB)]
            for h in ss:
                h.wait()

        plsc.subcore_barrier()
        pltpu.sync_copy(acc_sh.at[pl.ds(s * _SL, _SL)],
                        out_hbm.at[c, pl.ds(s * _SL, _SL)])

    return degree


_segsum = _make_segsum(F1)
_segsum2 = _make_segsum(F2)
_degree = _make_degree()


_FR = NPAD * F1 // 128
_FBR = 256
_FGRID = (_FR // _FBR,)


def _mm1_body(x_ref, w_ref, o_ref):
    o_ref[...] = jnp.dot(x_ref[...], w_ref[...],
                         preferred_element_type=jnp.float32)


def _scale_body(p_ref, h1_ref, sel_ref, dis_ref, h1s_ref, dis2_ref):
    deg = p_ref[0] + p_ref[1] + 1.0
    dis = lax.rsqrt(deg)
    dis_ref[...] = dis
    h1s_ref[...] = h1_ref[...] * dis
    d64 = dis.reshape(_FBR // 8, 8 * 128)
    dis2_ref[...] = jnp.dot(d64, sel_ref[...], preferred_element_type=jnp.float32)


def _layer1_body(p_ref, h1s_ref, dis_ref, b1_ref, bd2_ref, r_ref, v2_ref):
    dis = dis_ref[...]
    o1 = dis * (p_ref[0] + p_ref[1] + h1s_ref[...]) + b1_ref[...]
    r = dis * jnp.maximum(o1, 0.0)
    r_ref[...] = r
    r64 = r.reshape(_FBR // 8, 8 * 128)
    v2_ref[...] = jnp.dot(r64, bd2_ref[...], preferred_element_type=jnp.float32)


def _layer2_body(p_ref, v2_ref, dis2_ref, b2_ref, kp_ref, kd_ref, o_ref):
    o2 = dis2_ref[...] * (p_ref[0] + p_ref[1] + v2_ref[...]) + b2_ref[...]
    ps = jnp.dot(o2, kp_ref[...], preferred_element_type=jnp.float32)
    pd = jnp.dot(o2, kd_ref[...], preferred_element_type=jnp.float32)
    m = 0.5 * (ps + jnp.abs(pd))
    es = jnp.dot(jnp.exp(o2 - m), kp_ref[...], preferred_element_type=jnp.float32)
    o_ref[...] = o2 - (m + jnp.log(es))


def _flat_spec(width=128):
    return pl.BlockSpec((_FBR, width), lambda i: (i, 0))


def _pairflat_spec():
    return pl.BlockSpec((_NC, _FBR, 128), lambda i: (0, i, 0))


def _const_spec(shape):
    return pl.BlockSpec(shape, lambda i: tuple(0 for _ in shape))


_FLAT_SDS = jax.ShapeDtypeStruct((_FR, 128), jnp.float32)


def kernel(x, edge_index, W1, b1, W2, b2):
    ei = edge_index.astype(jnp.int32)
    src = ei[0].reshape(ROWS, EW)
    dst = ei[1].reshape(ROWS, EW)
    x_pad = jnp.pad(x, ((0, NPAD - N), (0, 0)))
    zeros = jnp.zeros((NPAD, F1), jnp.float32)
    ones = jnp.ones((NPAD, F1), jnp.float32)

    eye8 = jnp.eye(8, dtype=jnp.float32)
    eye64 = jnp.eye(64, dtype=jnp.float32)
    bd1 = jnp.kron(eye8, W1)
    bd2 = jnp.kron(eye64, W2.astype(jnp.float32))
    sel2 = jnp.kron(eye64, jnp.eye(F1, dtype=jnp.float32)[:, :F2])
    kp = jnp.kron(eye64, jnp.ones((F2, F2), jnp.float32))
    kd = jnp.kron(eye64, jnp.array([[1., -1.], [-1., 1.]], jnp.float32))
    b1t = jnp.tile(b1, 8).reshape(1, 128)
    b2t = jnp.tile(b2, 64).reshape(1, 128)
    zeros2 = jnp.zeros((NPAD, F2), jnp.float32)

    degp = _degree(ones, dst, zeros)
    degp_f = degp.reshape(_NC, _FR, 128)

    h1_f = pl.pallas_call(
        _mm1_body,
        grid=_FGRID,
        in_specs=[pl.BlockSpec((_FBR, 8 * F0), lambda i: (i, 0)),
                  _const_spec((8 * F0, 128))],
        out_specs=_flat_spec(),
        out_shape=_FLAT_SDS,
    )(x_pad.reshape(_FR, 8 * F0), bd1)

    dis_f, h1s_f, dis2_f = pl.pallas_call(
        _scale_body,
        grid=_FGRID,
        in_specs=[_pairflat_spec(), _flat_spec(), _const_spec((8 * 128, 128))],
        out_specs=[_flat_spec(), _flat_spec(),
                   pl.BlockSpec((_FBR // 8, 128), lambda i: (i, 0))],
        out_shape=[_FLAT_SDS, _FLAT_SDS,
                   jax.ShapeDtypeStruct((_FR // 8, 128), jnp.float32)],
    )(degp_f, h1_f, sel2)

    a1p = _segsum(h1s_f.reshape(NPAD, F1), src, dst, zeros)

    r_f, v2_f = pl.pallas_call(
        _layer1_body,
        grid=_FGRID,
        in_specs=[_pairflat_spec(), _flat_spec(), _flat_spec(),
                  _const_spec((1, 128)), _const_spec((8 * 128, 128))],
        out_specs=[_flat_spec(),
                   pl.BlockSpec((_FBR // 8, 128), lambda i: (i, 0))],
        out_shape=[_FLAT_SDS,
                   jax.ShapeDtypeStruct((_FR // 8, 128), jnp.float32)],
    )(a1p.reshape(_NC, _FR, 128), h1s_f, dis_f, b1t, bd2)

    a2p = _segsum2(v2_f.reshape(NPAD, F2), src, dst, zeros2)

    out_f = pl.pallas_call(
        _layer2_body,
        grid=(1,),
        in_specs=[pl.BlockSpec((_NC, _FR // 8, 128), lambda i: (0, 0, 0)),
                  pl.BlockSpec((_FR // 8, 128), lambda i: (0, 0)),
                  pl.BlockSpec((_FR // 8, 128), lambda i: (0, 0)),
                  _const_spec((1, 128)),
                  _const_spec((128, 128)), _const_spec((128, 128))],
        out_specs=pl.BlockSpec((_FR // 8, 128), lambda i: (0, 0)),
        out_shape=jax.ShapeDtypeStruct((_FR // 8, 128), jnp.float32),
    )(a2p.reshape(_NC, _FR // 8, 128), v2_f, dis2_f, b2t, kp, kd)

    return out_f.reshape(NPAD, F2)[:N]


# speedup vs baseline: 1.2801x; 1.2801x over previous
import functools

import jax
import jax.numpy as jnp
from jax import lax
from jax.experimental import pallas as pl
from jax.experimental.pallas import tpu as pltpu
from jax.experimental.pallas import tpu_sc as plsc

N = 10000
F0 = 128
F1 = 16
F2 = 2
E = 320000

NPAD = 10240
EW = 125
ROWS = E // EW

_NC = 2
_NS = 16
_RPT = ROWS // (_NC * _NS)
_SL = NPAD // _NS

_BR = 1024
_NB = 8


_MESH = plsc.VectorSubcoreMesh(core_axis_name="c", subcore_axis_name="s")
_SC_PARAMS = pltpu.CompilerParams(use_tc_tiling_on_sc=False)


def _make_segsum():
    @functools.partial(
        pl.kernel,
        out_type=jax.ShapeDtypeStruct((_NC, NPAD, F1), jnp.float32),
        mesh=_MESH,
        compiler_params=_SC_PARAMS,
        scratch_types=[
            pltpu.VMEM((_RPT, EW), jnp.int32),
            pltpu.VMEM((_RPT, EW), jnp.int32),
            pltpu.VMEM((_NB, EW, F1), jnp.float32),
            pltpu.VMEM_SHARED((NPAD, F1), jnp.float32),
            pltpu.VMEM_SHARED((NPAD, F1), jnp.float32),
            pltpu.SemaphoreType.DMA,
            pltpu.SemaphoreType.DMA,
        ],
    )
    def segsum(vals_hbm, src_hbm, dst_hbm, zeros_hbm, out_hbm,
               src_v, dst_v, rows_v, vals_sh, acc_sh, gsem, ssem):
        c = lax.axis_index("c")
        s = lax.axis_index("s")
        wid = s * _NC + c
        pltpu.sync_copy(src_hbm.at[pl.ds(wid * _RPT, _RPT)], src_v)
        pltpu.sync_copy(dst_hbm.at[pl.ds(wid * _RPT, _RPT)], dst_v)
        pltpu.sync_copy(vals_hbm.at[pl.ds(s * _SL, _SL)],
                        vals_sh.at[pl.ds(s * _SL, _SL)])
        pltpu.sync_copy(zeros_hbm.at[pl.ds(s * _SL, _SL)],
                        acc_sh.at[pl.ds(s * _SL, _SL)])
        plsc.subcore_barrier()

        @pl.loop(0, _RPT, step=_NB)
        def _(j0):
            gs = [pltpu.async_copy(vals_sh.at[src_v.at[j0 + b]],
                                   rows_v.at[b], gsem)
                  for b in range(_NB)]
            ss = []
            for b in range(_NB):
                gs[b].wait()
                ss.append(pltpu.async_copy(rows_v.at[b],
                                           acc_sh.at[dst_v.at[j0 + b]],
                                           ssem, add=True))
            for h in ss:
                h.wait()

        plsc.subcore_barrier()
        pltpu.sync_copy(acc_sh.at[pl.ds(s * _SL, _SL)],
                        out_hbm.at[c, pl.ds(s * _SL, _SL)])

    return segsum


def _make_degree():
    @functools.partial(
        pl.kernel,
        out_type=jax.ShapeDtypeStruct((_NC, NPAD, F1), jnp.float32),
        mesh=_MESH,
        compiler_params=_SC_PARAMS,
        scratch_types=[
            pltpu.VMEM((_RPT, EW), jnp.int32),
            pltpu.VMEM((EW, F1), jnp.float32),
            pltpu.VMEM_SHARED((NPAD, F1), jnp.float32),
            pltpu.SemaphoreType.DMA,
        ],
    )
    def degree(ones_hbm, dst_hbm, zeros_hbm, out_hbm,
               dst_v, ones_v, acc_sh, ssem):
        c = lax.axis_index("c")
        s = lax.axis_index("s")
        wid = s * _NC + c
        pltpu.sync_copy(dst_hbm.at[pl.ds(wid * _RPT, _RPT)], dst_v)
        pltpu.sync_copy(ones_hbm.at[pl.ds(0, EW)], ones_v)
        pltpu.sync_copy(zeros_hbm.at[pl.ds(s * _SL, _SL)],
                        acc_sh.at[pl.ds(s * _SL, _SL)])
        plsc.subcore_barrier()

        @pl.loop(0, _RPT, step=_NB)
        def _(j0):
            ss = [pltpu.async_copy(ones_v, acc_sh.at[dst_v.at[j0 + b]],
                                   ssem, add=True)
                  for b in range(_NB)]
            for h in ss:
                h.wait()

        plsc.subcore_barrier()
        pltpu.sync_copy(acc_sh.at[pl.ds(s * _SL, _SL)],
                        out_hbm.at[c, pl.ds(s * _SL, _SL)])

    return degree


_segsum = _make_segsum()
_degree = _make_degree()


_FR = NPAD * F1 // 128
_FBR = 256
_FGRID = (_FR // _FBR,)


def _mm1_body(x_ref, w_ref, o_ref):
    o_ref[...] = jnp.dot(x_ref[...], w_ref[...],
                         preferred_element_type=jnp.float32)


def _scale_body(p_ref, h1_ref, dis_ref, h1s_ref):
    deg = p_ref[0] + p_ref[1] + 1.0
    dis = lax.rsqrt(deg)
    dis_ref[...] = dis
    h1s_ref[...] = h1_ref[...] * dis


def _layer1_body(p_ref, h1s_ref, dis_ref, b1_ref, r_ref):
    dis = dis_ref[...]
    o1 = dis * (p_ref[0] + p_ref[1] + h1s_ref[...]) + b1_ref[...]
    r_ref[...] = dis * jnp.maximum(o1, 0.0)


def _layer2_body(p_ref, r_ref, dis_ref, bd2_ref, b2_ref, kp_ref, kd_ref, o_ref):
    t = dis_ref[...] * (p_ref[0] + p_ref[1] + r_ref[...])
    o2 = jnp.dot(t, bd2_ref[...], preferred_element_type=jnp.float32) + b2_ref[...]
    ps = jnp.dot(o2, kp_ref[...], preferred_element_type=jnp.float32)
    pd = jnp.dot(o2, kd_ref[...], preferred_element_type=jnp.float32)
    m = 0.5 * (ps + jnp.abs(pd))
    es = jnp.dot(jnp.exp(o2 - m), kp_ref[...], preferred_element_type=jnp.float32)
    o_ref[...] = o2 - (m + jnp.log(es))


def _flat_spec(width=128):
    return pl.BlockSpec((_FBR, width), lambda i: (i, 0))


def _pairflat_spec():
    return pl.BlockSpec((_NC, _FBR, 128), lambda i: (0, i, 0))


def _const_spec(shape):
    return pl.BlockSpec(shape, lambda i: tuple(0 for _ in shape))


_FLAT_SDS = jax.ShapeDtypeStruct((_FR, 128), jnp.float32)


def kernel(x, edge_index, W1, b1, W2, b2):
    ei = edge_index.astype(jnp.int32)
    src = ei[0].reshape(ROWS, EW)
    dst = ei[1].reshape(ROWS, EW)
    x_pad = jnp.pad(x, ((0, NPAD - N), (0, 0)))
    zeros = jnp.zeros((NPAD, F1), jnp.float32)
    ones = jnp.ones((NPAD, F1), jnp.float32)

    eye8 = jnp.eye(8, dtype=jnp.float32)
    bd1 = jnp.kron(eye8, W1)
    bd2 = jnp.kron(eye8, W2.astype(jnp.float32))
    kp = jnp.kron(eye8, jnp.ones((F2, F2), jnp.float32))
    kd = jnp.kron(eye8, jnp.array([[1., -1.], [-1., 1.]], jnp.float32))
    b1t = jnp.tile(b1, 8).reshape(1, 128)
    b2t = jnp.tile(b2, 8).reshape(1, F1)

    degp = _degree(ones, dst, zeros)
    degp_f = degp.reshape(_NC, _FR, 128)

    h1_f = pl.pallas_call(
        _mm1_body,
        grid=_FGRID,
        in_specs=[pl.BlockSpec((_FBR, 8 * F0), lambda i: (i, 0)),
                  _const_spec((8 * F0, 128))],
        out_specs=_flat_spec(),
        out_shape=_FLAT_SDS,
    )(x_pad.reshape(_FR, 8 * F0), bd1)

    dis_f, h1s_f = pl.pallas_call(
        _scale_body,
        grid=_FGRID,
        in_specs=[_pairflat_spec(), _flat_spec()],
        out_specs=[_flat_spec(), _flat_spec()],
        out_shape=[_FLAT_SDS, _FLAT_SDS],
    )(degp_f, h1_f)

    a1p = _segsum(h1s_f.reshape(NPAD, F1), src, dst, zeros)

    r_f = pl.pallas_call(
        _layer1_body,
        grid=_FGRID,
        in_specs=[_pairflat_spec(), _flat_spec(), _flat_spec(),
                  _const_spec((1, 128))],
        out_specs=_flat_spec(),
        out_shape=_FLAT_SDS,
    )(a1p.reshape(_NC, _FR, 128), h1s_f, dis_f, b1t)

    a2p = _segsum(r_f.reshape(NPAD, F1), src, dst, zeros)

    out_f = pl.pallas_call(
        _layer2_body,
        grid=_FGRID,
        in_specs=[_pairflat_spec(), _flat_spec(), _flat_spec(),
                  _const_spec((128, F1)), _const_spec((1, F1)),
                  _const_spec((F1, F1)), _const_spec((F1, F1))],
        out_specs=pl.BlockSpec((_FBR, F1), lambda i: (i, 0)),
        out_shape=jax.ShapeDtypeStruct((_FR, F1), jnp.float32),
    )(a2p.reshape(_NC, _FR, 128), r_f, dis_f, bd2, b2t, kp, kd)

    return out_f.reshape(NPAD, F2)[:N]
